# baseline (speedup 1.0000x reference)
import functools

import jax
import jax.numpy as jnp
from jax import lax
from jax.experimental import pallas as pl
from jax.experimental.pallas import tpu as pltpu
from jax.experimental.pallas import tpu_sc as plsc

N = 10000
E = 160000
D = 256
H = 8
HD = 32
FF = 512
B = 64
SCALE = HD ** -0.5

NC = 2
NS = 16
NW = NC * NS
NPH = 2
TROWS = 160
NL = NW * NPH
NOUT = NL * TROWS
ECHUNK = E // NS
LB = 1024
CAP = 4128
CE = 64
DW = D // 2

def _sc_mesh():
    return plsc.VectorSubcoreMesh(core_axis_name="c", subcore_axis_name="s",
                                  num_cores=NC, num_subcores=NS)


@functools.cache
def _edge_partition_kernel():
    return pl.kernel(
        _edge_partition_body,
        out_type=(
            jax.ShapeDtypeStruct((NL * CAP,), jnp.int32),
            jax.ShapeDtypeStruct((NL * CAP,), jnp.int32),
        ),
        mesh=_sc_mesh(),
        scratch_types=[
            pltpu.VMEM((ECHUNK,), jnp.int32),
            pltpu.VMEM((ECHUNK,), jnp.int32),
            pltpu.VMEM((CAP,), jnp.int32),
            pltpu.VMEM((CAP,), jnp.int32),
            pltpu.VMEM((CAP,), jnp.int32),
            pltpu.VMEM((CAP,), jnp.int32),
        ],
        compiler_params=pltpu.CompilerParams(needs_layout_passes=False),
    )


def _edge_partition(dst, src):
    return _edge_partition_kernel()(dst, src)


def _edge_partition_body(dst_hbm, src_hbm, dloc_out, gsrc_out,
                         dbuf, sbuf, od0, os0, od1, os1):
    c = lax.axis_index("c")
    s = lax.axis_index("s")
    w = c * NS + s

    pad_d = jnp.full((16,), TROWS, dtype=jnp.int32)
    pad_s = jnp.zeros((16,), dtype=jnp.int32)

    def prefill(i, _):
        od0[pl.ds(i * 16, 16)] = pad_d
        os0[pl.ds(i * 16, 16)] = pad_s
        od1[pl.ds(i * 16, 16)] = pad_d
        os1[pl.ds(i * 16, 16)] = pad_s
        return 0

    lax.fori_loop(0, CAP // 16, prefill, 0)

    lo0 = w * TROWS
    lo1 = NW * TROWS + w * TROWS

    def outer(ch, curs):
        pltpu.sync_copy(dst_hbm.at[pl.ds(ch * ECHUNK, ECHUNK)], dbuf)
        pltpu.sync_copy(src_hbm.at[pl.ds(ch * ECHUNK, ECHUNK)], sbuf)

        def scan(i, curs):
            cur0, cur1 = curs
            dv = dbuf[pl.ds(i * 16, 16)]
            sv = sbuf[pl.ds(i * 16, 16)]
            m0 = (dv >= lo0) & (dv < lo0 + TROWS)
            cs0 = plsc.cumsum(m0.astype(jnp.int32))
            pos0 = jnp.minimum(cur0 + cs0 - 1, CAP - 17)
            plsc.store_scatter(od0, [pos0], dv - lo0, mask=m0)
            plsc.store_scatter(os0, [pos0], sv, mask=m0)
            m1 = (dv >= lo1) & (dv < lo1 + TROWS)
            cs1 = plsc.cumsum(m1.astype(jnp.int32))
            pos1 = jnp.minimum(cur1 + cs1 - 1, CAP - 17)
            plsc.store_scatter(od1, [pos1], dv - lo1, mask=m1)
            plsc.store_scatter(os1, [pos1], sv, mask=m1)
            return (cur0 + cs0[15], cur1 + cs1[15])

        return lax.fori_loop(0, ECHUNK // 16, scan, curs)

    t0, t1 = lax.fori_loop(0, NS, outer, (jnp.int32(0), jnp.int32(0)))
    t0 = jnp.minimum(t0, CAP - 32)
    t1 = jnp.minimum(t1, CAP - 32)

    od0[pl.ds(CAP - 16, 16)] = jnp.full((16,), t0, dtype=jnp.int32)
    od1[pl.ds(CAP - 16, 16)] = jnp.full((16,), t1, dtype=jnp.int32)
    pltpu.sync_copy(od0, dloc_out.at[pl.ds(w * CAP, CAP)])
    pltpu.sync_copy(os0, gsrc_out.at[pl.ds(w * CAP, CAP)])
    pltpu.sync_copy(od1, dloc_out.at[pl.ds((NW + w) * CAP, CAP)])
    pltpu.sync_copy(os1, gsrc_out.at[pl.ds((NW + w) * CAP, CAP)])


@functools.cache
def _edge_attention_kernel():
    return pl.kernel(
        _edge_attention_body,
        out_type=(
            jax.ShapeDtypeStruct((NOUT, D), jnp.float32),
            jax.ShapeDtypeStruct((NOUT, 16), jnp.float32),
        ),
        mesh=_sc_mesh(),
        scratch_types=[
            [pltpu.VMEM((CE,), jnp.int32)] * 2,
            [pltpu.VMEM((CE,), jnp.int32)] * 2,
            [pltpu.VMEM((CE,), jnp.int32)] * 2,
            [pltpu.VMEM((CE,), jnp.int32)] * 2,
            [pltpu.VMEM((CE,), jnp.int32)] * 2,
            [pltpu.VMEM((CE, DW), jnp.int32)] * 2,
            [pltpu.VMEM((CE, 2 * DW), jnp.int32)] * 2,
            pltpu.VMEM((TROWS, D), jnp.float32),
            pltpu.VMEM((TROWS, 16), jnp.float32),
            [pltpu.SemaphoreType.DMA] * 2,
            [pltpu.SemaphoreType.DMA] * 2,
        ],
        compiler_params=pltpu.CompilerParams(needs_layout_passes=False),
    )


def _edge_attention(q, kv, dloc, gsrc):
    return _edge_attention_kernel()(q, kv, dloc, gsrc)


def _edge_attention_body(q_hbm, kv_hbm, dloc_hbm, gsrc_hbm,
                         num_out, den_out,
                         dchunk, schunk, didx, qidx, sidx, qrows, kvrows,
                         nacc, dacc, semg, seml):
    c = lax.axis_index("c")
    s = lax.axis_index("s")
    w = c * NS + s
    zv = jnp.zeros((16,), dtype=jnp.float32)
    iota16 = lax.iota(jnp.int32, 16)

    def lbase(j):
        return jnp.minimum(j * CE, CAP - CE)

    def phase(p, _):
        lrow = p * NW + w

        pltpu.sync_copy(dloc_hbm.at[pl.ds(lrow * CAP + CAP - 16, 16)],
                        qidx[0].at[pl.ds(0, 16)])
        n_e = qidx[0][pl.ds(0, 16)][0]

        def znum(i, _):
            nacc[i // 16, pl.ds((i % 16) * 16, 16)] = zv
            return 0

        def zden(i, _):
            dacc[i, pl.ds(0, 16)] = zv
            return 0

        lax.fori_loop(0, TROWS * 16, znum, 0)
        lax.fori_loop(0, TROWS, zden, 0)

        qoff = lrow * TROWS

        def fire_lists(j, b):
            bj = lrow * CAP + lbase(j)
            pltpu.async_copy(dloc_hbm.at[pl.ds(bj, CE)], dchunk[b], seml[b])
            pltpu.async_copy(gsrc_hbm.at[pl.ds(bj, CE)], schunk[b], seml[b])

        def wait_lists(j, b):
            bj = lrow * CAP + lbase(j)
            pltpu.make_async_copy(dloc_hbm.at[pl.ds(bj, CE)], dchunk[b],
                                  seml[b]).wait()
            pltpu.make_async_copy(gsrc_hbm.at[pl.ds(bj, CE)], schunk[b],
                                  seml[b]).wait()

        def build_idx(b):
            for t in range(CE // 16):
                dv = jnp.minimum(dchunk[b][pl.ds(t * 16, 16)], TROWS - 1)
                sv = schunk[b][pl.ds(t * 16, 16)]
                didx[b][pl.ds(t * 16, 16)] = dv
                qidx[b][pl.ds(t * 16, 16)] = jnp.minimum(dv + qoff, N - 1)
                sidx[b][pl.ds(t * 16, 16)] = sv

        def fire_gathers(b):
            pltpu.async_copy(q_hbm.at[qidx[b]], qrows[b], semg[b])
            pltpu.async_copy(kv_hbm.at[sidx[b]], kvrows[b], semg[b])

        def wait_gathers(b):
            pltpu.make_async_copy(q_hbm.at[qidx[b]], qrows[b], semg[b]).wait()
            pltpu.make_async_copy(kv_hbm.at[sidx[b]], kvrows[b],
                                  semg[b]).wait()

        mhi = jnp.int32(-65536)

        def unlo(wv):
            return lax.bitcast_convert_type(wv << 16, jnp.float32)

        def unhi(wv):
            return lax.bitcast_convert_type(wv & mhi, jnp.float32)

        def compute(i, b):
            base = i * CE

            def group(t, _):
                dloc16 = didx[b][pl.ds(t * 16, 16)]
                for r in range(16):
                    e = t * 16 + r
                    ex = jnp.zeros((16,), dtype=jnp.float32)
                    for h in range(H):
                        qw = qrows[b][e, pl.ds(h * 16, 16)]
                        kw = kvrows[b][e, pl.ds(h * 16, 16)]
                        pr = unlo(qw) * unlo(kw) + unhi(qw) * unhi(kw)
                        sh = jnp.sum(pr) * SCALE
                        ex = jnp.where(iota16 == h,
                                       jnp.full((16,), sh, dtype=jnp.float32),
                                       ex)
                    valid = ((base + t * 16 + r) < n_e).astype(jnp.float32)
                    ev = jnp.exp(ex) * valid
                    rowv = jnp.full((16,), dloc16[r], dtype=jnp.int32)
                    plsc.addupdate_scatter(dacc, [rowv, iota16], ev)
                    for h in range(H):
                        a = jnp.full((16,), ev[h], dtype=jnp.float32)
                        vw = kvrows[b][e, pl.ds(DW + h * 16, 16)]
                        col = h * HD + 2 * iota16
                        plsc.addupdate_scatter(
                            nacc, [rowv, col], unlo(vw) * a)
                        plsc.addupdate_scatter(
                            nacc, [rowv, col + 1], unhi(vw) * a)
                return 0

            lax.fori_loop(0, CE // 16, group, 0)

        fire_lists(0, 0)
        wait_lists(0, 0)
        build_idx(0)
        fire_gathers(0)
        fire_lists(1, 1)

        nchp = ((n_e + (CE - 1)) // CE + 1) // 2

        def pair(i2, _):
            for b in range(2):
                i = 2 * i2 + b
                bn = 1 - b
                wait_gathers(b)
                wait_lists(i + 1, bn)
                build_idx(bn)
                fire_gathers(bn)
                fire_lists(i + 2, b)
                compute(i, b)
            return 0

        lax.fori_loop(0, nchp, pair, 0)

        wait_gathers(0)
        wait_lists(2 * nchp + 1, 1)

        pltpu.sync_copy(nacc, num_out.at[pl.ds(qoff, TROWS)])
        pltpu.sync_copy(dacc, den_out.at[pl.ds(qoff, TROWS)])
        return 0

    lax.fori_loop(0, NPH, phase, 0)


SROWS = 320


@functools.cache
def _batch_max_kernel():
    return pl.kernel(
        _batch_max_body,
        out_type=jax.ShapeDtypeStruct((NW, B, D), jnp.float32),
        mesh=_sc_mesh(),
        scratch_types=[
            pltpu.VMEM((16, D), jnp.float32),
            pltpu.VMEM((16,), jnp.int32),
            pltpu.VMEM((B, D), jnp.float32),
        ],
        compiler_params=pltpu.CompilerParams(needs_layout_passes=False),
    )


def _batch_max(h, bid):
    return _batch_max_kernel()(h, bid)


def _batch_max_body(h_hbm, bid_hbm, part_out, rowbuf, bbuf, acc):
    c = lax.axis_index("c")
    s = lax.axis_index("s")
    w = c * NS + s
    n0 = w * SROWS
    nr = jnp.clip(N - n0, 0, SROWS)

    ninf = jnp.full((16,), -jnp.inf, dtype=jnp.float32)

    def zacc(i, _):
        acc[i // 16, pl.ds((i % 16) * 16, 16)] = ninf
        return 0

    lax.fori_loop(0, B * 16, zacc, 0)

    def chunkfn(ci, _):
        base = n0 + ci * 16
        pltpu.sync_copy(h_hbm.at[pl.ds(base, 16)], rowbuf)
        pltpu.sync_copy(bid_hbm.at[pl.ds(base, 16)], bbuf)

        bv = bbuf[pl.ds(0, 16)]
        for r in range(16):
            bid = bv[r]
            for j in range(D // 16):
                cur = acc[bid, pl.ds(j * 16, 16)]
                acc[bid, pl.ds(j * 16, 16)] = jnp.maximum(
                    cur, rowbuf[r, pl.ds(j * 16, 16)])
        return 0

    lax.fori_loop(0, nr // 16, chunkfn, 0)
    pltpu.sync_copy(acc, part_out.at[w])


def _qkv0_body(x_ref, emb_ref, w_ref, h_ref, q_ref, kv_ref):
    xrow = x_ref[0]
    onehot_t = (lax.broadcasted_iota(jnp.int32, (128, ROWS), 0) == xrow
                ).astype(jnp.float32)
    h = lax.dot_general(onehot_t, emb_ref[...], (((0,), (0,)), ((), ())),
                        preferred_element_type=jnp.float32)
    qkv = jnp.dot(h, w_ref[...], preferred_element_type=jnp.float32)
    h_ref[...] = h
    q_ref[...] = qkv[:, D:2 * D].astype(jnp.bfloat16)
    kv_ref[:, :D] = qkv[:, :D].astype(jnp.bfloat16)
    kv_ref[:, D:] = qkv[:, 2 * D:].astype(jnp.bfloat16)


def _qkv1_body(h_ref, w_ref, q_ref, kv_ref):
    qkv = jnp.dot(h_ref[...], w_ref[...], preferred_element_type=jnp.float32)
    q_ref[...] = qkv[:, D:2 * D].astype(jnp.bfloat16)
    kv_ref[:, :D] = qkv[:, :D].astype(jnp.bfloat16)
    kv_ref[:, D:] = qkv[:, 2 * D:].astype(jnp.bfloat16)


def _post_body(h_ref, num_ref, den_ref, wo_ref, g1_ref, b1_ref, g2_ref,
               b2_ref, w1_ref, bf1_ref, w2_ref, bf2_ref, out_ref):
    rows = num_ref.shape[0]
    ih = lax.broadcasted_iota(jnp.int32, (16, D), 0)
    idd = lax.broadcasted_iota(jnp.int32, (16, D), 1)
    expand = (idd // HD == ih).astype(jnp.float32)
    den_rep = jnp.dot(den_ref[...], expand, preferred_element_type=jnp.float32)
    att = num_ref[...] / jnp.maximum(den_rep, 1e-16)
    att = jnp.dot(att, wo_ref[...], preferred_element_type=jnp.float32)
    h1 = h_ref[...] + att
    mu = h1.mean(-1, keepdims=True)
    var = ((h1 - mu) ** 2).mean(-1, keepdims=True)
    h1 = (h1 - mu) * lax.rsqrt(var + 1e-5) * g1_ref[...] + b1_ref[...]
    ff = jnp.maximum(
        jnp.dot(h1, w1_ref[...], preferred_element_type=jnp.float32)
        + bf1_ref[...], 0.0)
    ff = jnp.dot(ff, w2_ref[...], preferred_element_type=jnp.float32) \
        + bf2_ref[...]
    h2 = h1 + ff
    mu = h2.mean(-1, keepdims=True)
    var = ((h2 - mu) ** 2).mean(-1, keepdims=True)
    out_ref[...] = (h2 - mu) * lax.rsqrt(var + 1e-5) * g2_ref[...] \
        + b2_ref[...]


def _combine_body(part_ref, out_ref):
    acc = part_ref[0]
    for i in range(1, NW):
        acc = jnp.maximum(acc, part_ref[i])
    out_ref[...] = jnp.where(jnp.isfinite(acc), acc, 0.0)


ROWS = 400
GRID = N // ROWS


def _full(shape):
    return pl.BlockSpec(shape, lambda i: (0,) * len(shape))


def _rows(width):
    return pl.BlockSpec((ROWS, width), lambda i: (i, 0))


def _tc_qkv0(x3, emb, wcat):
    return pl.pallas_call(
        _qkv0_body,
        grid=(GRID,),
        in_specs=[
            pl.BlockSpec((1, 1, ROWS), lambda i: (i, 0, 0)),
            _full((128, D)),
            _full((D, 3 * D)),
        ],
        out_specs=[_rows(D), _rows(D), _rows(2 * D)],
        out_shape=[jax.ShapeDtypeStruct((N, D), jnp.float32),
                   jax.ShapeDtypeStruct((N, D), jnp.bfloat16),
                   jax.ShapeDtypeStruct((N, 2 * D), jnp.bfloat16)],
    )(x3, emb, wcat)


def _tc_qkv1(h, wcat):
    return pl.pallas_call(
        _qkv1_body,
        grid=(GRID,),
        in_specs=[_rows(D), _full((D, 3 * D))],
        out_specs=[_rows(D), _rows(2 * D)],
        out_shape=[jax.ShapeDtypeStruct((N, D), jnp.bfloat16),
                   jax.ShapeDtypeStruct((N, 2 * D), jnp.bfloat16)],
    )(h, wcat)


def _pack16(a):
    return lax.bitcast_convert_type(
        a.reshape(N, a.shape[1] // 2, 2), jnp.int32)


def _tc_post(h, num, den, wo, g1, b1, g2, b2, w1, bf1, w2, bf2):
    return pl.pallas_call(
        _post_body,
        grid=(GRID,),
        in_specs=[
            _rows(D), _rows(D), _rows(16), _full((D, D)),
            _full((1, D)), _full((1, D)), _full((1, D)), _full((1, D)),
            _full((D, FF)), _full((1, FF)), _full((FF, D)), _full((1, D)),
        ],
        out_specs=_rows(D),
        out_shape=jax.ShapeDtypeStruct((N, D), jnp.float32),
    )(h, num, den, wo, g1.reshape(1, D), b1.reshape(1, D),
      g2.reshape(1, D), b2.reshape(1, D), w1, bf1.reshape(1, FF), w2,
      bf2.reshape(1, D))


def _tc_combine(parts):
    return pl.pallas_call(
        _combine_body,
        out_shape=jax.ShapeDtypeStruct((B, D), jnp.float32),
    )(parts)


def _layer(h, src, dst_parts, wqk, wv, wo, g1, b1, g2, b2, w1, bf1, w2, bf2,
           x3=None, emb=None):
    dloc, gsrc = dst_parts
    wcat = jnp.concatenate([wqk, wv], axis=1)
    if x3 is not None:
        h, q, kv = _tc_qkv0(x3, emb, wcat)
    else:
        q, kv = _tc_qkv1(h, wcat)
    num, den = _edge_attention(_pack16(q), _pack16(kv), dloc, gsrc)
    return _tc_post(h, num[:N], den[:N], wo, g1, b1, g2, b2, w1, bf1, w2,
                    bf2)


def kernel(x, complete_edge_index, ptr, batch, emb,
           W_qk_0, W_v_0, W_o_0, ln1_g_0, ln1_b_0, ln2_g_0, ln2_b_0,
           W1_0, b1_0, W2_0, b2_0,
           W_qk_1, W_v_1, W_o_1, ln1_g_1, ln1_b_1, ln2_g_1, ln2_b_1,
           W1_1, b1_1, W2_1, b2_1):
    src = complete_edge_index[0].astype(jnp.int32)
    dst = complete_edge_index[1].astype(jnp.int32)
    parts = _edge_partition(dst, src)
    x3 = x.astype(jnp.int32).reshape(GRID, 1, ROWS)
    h = _layer(None, src, parts, W_qk_0, W_v_0, W_o_0, ln1_g_0, ln1_b_0,
               ln2_g_0, ln2_b_0, W1_0, b1_0, W2_0, b2_0, x3=x3, emb=emb)
    h = _layer(h, src, parts, W_qk_1, W_v_1, W_o_1, ln1_g_1, ln1_b_1,
               ln2_g_1, ln2_b_1, W1_1, b1_1, W2_1, b2_1)
    partials = _batch_max(h, batch.astype(jnp.int32))
    return _tc_combine(partials)


# speedup vs baseline: 1.6145x; 1.6145x over previous
import functools

import jax
import jax.numpy as jnp
from jax import lax
from jax.experimental import pallas as pl
from jax.experimental.pallas import tpu as pltpu
from jax.experimental.pallas import tpu_sc as plsc

N = 10000
E = 160000
D = 256
H = 8
HD = 32
FF = 512
B = 64
SCALE = HD ** -0.5

NC = 2
NS = 16
NW = NC * NS
NPH = 2
TROWS = 160
NL = NW * NPH
NOUT = NL * TROWS
ECHUNK = E // NS
LB = 1024
CAP = 4128
CE = 64
DW = D // 2

def _sc_mesh():
    return plsc.VectorSubcoreMesh(core_axis_name="c", subcore_axis_name="s",
                                  num_cores=NC, num_subcores=NS)


@functools.cache
def _edge_partition_kernel():
    return pl.kernel(
        _edge_partition_body,
        out_type=(
            jax.ShapeDtypeStruct((NL * CAP,), jnp.int32),
            jax.ShapeDtypeStruct((NL * CAP,), jnp.int32),
        ),
        mesh=_sc_mesh(),
        scratch_types=[
            pltpu.VMEM((ECHUNK,), jnp.int32),
            pltpu.VMEM((ECHUNK,), jnp.int32),
            pltpu.VMEM((CAP,), jnp.int32),
            pltpu.VMEM((CAP,), jnp.int32),
            pltpu.VMEM((CAP,), jnp.int32),
            pltpu.VMEM((CAP,), jnp.int32),
        ],
        compiler_params=pltpu.CompilerParams(needs_layout_passes=False),
    )


def _edge_partition(dst, src):
    return _edge_partition_kernel()(dst, src)


def _edge_partition_body(dst_hbm, src_hbm, dloc_out, gsrc_out,
                         dbuf, sbuf, od0, os0, od1, os1):
    c = lax.axis_index("c")
    s = lax.axis_index("s")
    w = c * NS + s

    pad_d = jnp.full((16,), TROWS, dtype=jnp.int32)
    pad_s = jnp.zeros((16,), dtype=jnp.int32)

    def prefill(i, _):
        od0[pl.ds(i * 16, 16)] = pad_d
        os0[pl.ds(i * 16, 16)] = pad_s
        od1[pl.ds(i * 16, 16)] = pad_d
        os1[pl.ds(i * 16, 16)] = pad_s
        return 0

    lax.fori_loop(0, CAP // 16, prefill, 0)

    lo0 = w * TROWS
    lo1 = NW * TROWS + w * TROWS

    def outer(ch, curs):
        pltpu.sync_copy(dst_hbm.at[pl.ds(ch * ECHUNK, ECHUNK)], dbuf)
        pltpu.sync_copy(src_hbm.at[pl.ds(ch * ECHUNK, ECHUNK)], sbuf)

        def scan(i, curs):
            cur0, cur1 = curs
            dv = dbuf[pl.ds(i * 16, 16)]
            sv = sbuf[pl.ds(i * 16, 16)]
            m0 = (dv >= lo0) & (dv < lo0 + TROWS)
            cs0 = plsc.cumsum(m0.astype(jnp.int32))
            pos0 = jnp.minimum(cur0 + cs0 - 1, CAP - 17)
            plsc.store_scatter(od0, [pos0], dv - lo0, mask=m0)
            plsc.store_scatter(os0, [pos0], sv, mask=m0)
            m1 = (dv >= lo1) & (dv < lo1 + TROWS)
            cs1 = plsc.cumsum(m1.astype(jnp.int32))
            pos1 = jnp.minimum(cur1 + cs1 - 1, CAP - 17)
            plsc.store_scatter(od1, [pos1], dv - lo1, mask=m1)
            plsc.store_scatter(os1, [pos1], sv, mask=m1)
            return (cur0 + cs0[15], cur1 + cs1[15])

        return lax.fori_loop(0, ECHUNK // 16, scan, curs)

    t0, t1 = lax.fori_loop(0, NS, outer, (jnp.int32(0), jnp.int32(0)))
    t0 = jnp.minimum(t0, CAP - 32)
    t1 = jnp.minimum(t1, CAP - 32)

    od0[pl.ds(CAP - 16, 16)] = jnp.full((16,), t0, dtype=jnp.int32)
    od1[pl.ds(CAP - 16, 16)] = jnp.full((16,), t1, dtype=jnp.int32)
    pltpu.sync_copy(od0, dloc_out.at[pl.ds(w * CAP, CAP)])
    pltpu.sync_copy(os0, gsrc_out.at[pl.ds(w * CAP, CAP)])
    pltpu.sync_copy(od1, dloc_out.at[pl.ds((NW + w) * CAP, CAP)])
    pltpu.sync_copy(os1, gsrc_out.at[pl.ds((NW + w) * CAP, CAP)])


@functools.cache
def _edge_attention_kernel():
    return pl.kernel(
        _edge_attention_body,
        out_type=(
            jax.ShapeDtypeStruct((NOUT, D), jnp.float32),
            jax.ShapeDtypeStruct((NOUT, 16), jnp.float32),
        ),
        mesh=_sc_mesh(),
        scratch_types=[
            [pltpu.VMEM((CE,), jnp.int32)] * 2,
            [pltpu.VMEM((CE,), jnp.int32)] * 2,
            [pltpu.VMEM((CE,), jnp.int32)] * 2,
            [pltpu.VMEM((CE,), jnp.int32)] * 2,
            [pltpu.VMEM((CE,), jnp.int32)] * 2,
            [pltpu.VMEM((CE, DW), jnp.int32)] * 2,
            [pltpu.VMEM((CE, 2 * DW), jnp.int32)] * 2,
            pltpu.VMEM((CE, 16), jnp.float32),
            pltpu.VMEM((TROWS + 8, D), jnp.float32),
            pltpu.VMEM((TROWS + 8, 16), jnp.float32),
            [pltpu.SemaphoreType.DMA] * 2,
            [pltpu.SemaphoreType.DMA] * 2,
        ],
        compiler_params=pltpu.CompilerParams(needs_layout_passes=False),
    )


def _edge_attention(q, kv, dloc, gsrc):
    return _edge_attention_kernel()(q, kv, dloc, gsrc)


def _edge_attention_body(q_hbm, kv_hbm, dloc_hbm, gsrc_hbm,
                         num_out, den_out,
                         dchunk, schunk, didx, qidx, sidx, qrows, kvrows,
                         exbuf, nacc, dacc, semg, seml):
    c = lax.axis_index("c")
    s = lax.axis_index("s")
    w = c * NS + s
    zv = jnp.zeros((16,), dtype=jnp.float32)
    iota16 = lax.iota(jnp.int32, 16)

    def lbase(j):
        return jnp.minimum(j * CE, CAP - CE)

    def phase(p, _):
        lrow = p * NW + w

        pltpu.sync_copy(dloc_hbm.at[pl.ds(lrow * CAP + CAP - 16, 16)],
                        qidx[0].at[pl.ds(0, 16)])
        n_e = qidx[0][pl.ds(0, 16)][0]

        def znum(i, _):
            nacc[i // 16, pl.ds((i % 16) * 16, 16)] = zv
            return 0

        def zden(i, _):
            dacc[i, pl.ds(0, 16)] = zv
            return 0

        lax.fori_loop(0, (TROWS + 8) * 16, znum, 0)
        lax.fori_loop(0, TROWS + 8, zden, 0)

        qoff = lrow * TROWS

        def fire_lists(j, b):
            bj = lrow * CAP + lbase(j)
            pltpu.async_copy(dloc_hbm.at[pl.ds(bj, CE)], dchunk[b], seml[b])
            pltpu.async_copy(gsrc_hbm.at[pl.ds(bj, CE)], schunk[b], seml[b])

        def wait_lists(j, b):
            bj = lrow * CAP + lbase(j)
            pltpu.make_async_copy(dloc_hbm.at[pl.ds(bj, CE)], dchunk[b],
                                  seml[b]).wait()
            pltpu.make_async_copy(gsrc_hbm.at[pl.ds(bj, CE)], schunk[b],
                                  seml[b]).wait()

        def build_idx(b):
            for t in range(CE // 16):
                dv = jnp.minimum(dchunk[b][pl.ds(t * 16, 16)], TROWS)
                sv = schunk[b][pl.ds(t * 16, 16)]
                didx[b][pl.ds(t * 16, 16)] = dv
                qidx[b][pl.ds(t * 16, 16)] = jnp.minimum(dv + qoff, N - 1)
                sidx[b][pl.ds(t * 16, 16)] = sv

        def fire_gathers(b):
            pltpu.async_copy(q_hbm.at[qidx[b]], qrows[b], semg[b])
            pltpu.async_copy(kv_hbm.at[sidx[b]], kvrows[b], semg[b])

        def wait_gathers(b):
            pltpu.make_async_copy(q_hbm.at[qidx[b]], qrows[b], semg[b]).wait()
            pltpu.make_async_copy(kv_hbm.at[sidx[b]], kvrows[b],
                                  semg[b]).wait()

        mhi = jnp.int32(-65536)

        def unlo(wv):
            return lax.bitcast_convert_type(wv << 16, jnp.float32)

        def unhi(wv):
            return lax.bitcast_convert_type(wv & mhi, jnp.float32)

        def compute(i, b):
            def edge(e, _):
                ex = jnp.zeros((16,), dtype=jnp.float32)
                for h in range(H):
                    qw = qrows[b][e, pl.ds(h * 16, 16)]
                    kw = kvrows[b][e, pl.ds(h * 16, 16)]
                    pr = unlo(qw) * unlo(kw) + unhi(qw) * unhi(kw)
                    sh = jnp.sum(pr)
                    ex = jnp.where(iota16 == h,
                                   jnp.full((16,), sh, dtype=jnp.float32),
                                   ex)
                exbuf[e, pl.ds(0, 16)] = jnp.exp(ex)
                return 0

            lax.fori_loop(0, CE, edge, 0)

            def wgroup(t, _):
                dloc16 = didx[b][pl.ds(t * 16, 16)]
                for r in range(16):
                    e = t * 16 + r
                    ev = exbuf[e, pl.ds(0, 16)]
                    rowv = jnp.full((16,), dloc16[r], dtype=jnp.int32)
                    plsc.addupdate_scatter(dacc, [rowv, iota16], ev)
                    for h in range(H):
                        a = jnp.full((16,), ev[h], dtype=jnp.float32)
                        vw = kvrows[b][e, pl.ds(DW + h * 16, 16)]
                        col = h * HD + 2 * iota16
                        plsc.addupdate_scatter(
                            nacc, [rowv, col], unlo(vw) * a)
                        plsc.addupdate_scatter(
                            nacc, [rowv, col + 1], unhi(vw) * a)
                return 0

            lax.fori_loop(0, CE // 16, wgroup, 0)

        fire_lists(0, 0)
        wait_lists(0, 0)
        build_idx(0)
        fire_gathers(0)
        fire_lists(1, 1)

        nchp = ((n_e + (CE - 1)) // CE + 1) // 2

        def pair(i2, _):
            for b in range(2):
                i = 2 * i2 + b
                bn = 1 - b
                wait_gathers(b)
                wait_lists(i + 1, bn)
                build_idx(bn)
                fire_gathers(bn)
                fire_lists(i + 2, b)
                compute(i, b)
            return 0

        lax.fori_loop(0, nchp, pair, 0)

        wait_gathers(0)
        wait_lists(2 * nchp + 1, 1)

        pltpu.sync_copy(nacc.at[pl.ds(0, TROWS)],
                        num_out.at[pl.ds(qoff, TROWS)])
        pltpu.sync_copy(dacc.at[pl.ds(0, TROWS)],
                        den_out.at[pl.ds(qoff, TROWS)])
        return 0

    lax.fori_loop(0, NPH, phase, 0)


SROWS = 320


@functools.cache
def _batch_max_kernel():
    return pl.kernel(
        _batch_max_body,
        out_type=jax.ShapeDtypeStruct((NW, B, D), jnp.float32),
        mesh=_sc_mesh(),
        scratch_types=[
            pltpu.VMEM((16, D), jnp.float32),
            pltpu.VMEM((16,), jnp.int32),
            pltpu.VMEM((B, D), jnp.float32),
        ],
        compiler_params=pltpu.CompilerParams(needs_layout_passes=False),
    )


def _batch_max(h, bid):
    return _batch_max_kernel()(h, bid)


def _batch_max_body(h_hbm, bid_hbm, part_out, rowbuf, bbuf, acc):
    c = lax.axis_index("c")
    s = lax.axis_index("s")
    w = c * NS + s
    n0 = w * SROWS
    nr = jnp.clip(N - n0, 0, SROWS)

    ninf = jnp.full((16,), -jnp.inf, dtype=jnp.float32)

    def zacc(i, _):
        acc[i // 16, pl.ds((i % 16) * 16, 16)] = ninf
        return 0

    lax.fori_loop(0, B * 16, zacc, 0)

    def chunkfn(ci, _):
        base = n0 + ci * 16
        pltpu.sync_copy(h_hbm.at[pl.ds(base, 16)], rowbuf)
        pltpu.sync_copy(bid_hbm.at[pl.ds(base, 16)], bbuf)

        bv = bbuf[pl.ds(0, 16)]
        for r in range(16):
            bid = bv[r]
            for j in range(D // 16):
                cur = acc[bid, pl.ds(j * 16, 16)]
                acc[bid, pl.ds(j * 16, 16)] = jnp.maximum(
                    cur, rowbuf[r, pl.ds(j * 16, 16)])
        return 0

    lax.fori_loop(0, nr // 16, chunkfn, 0)
    pltpu.sync_copy(acc, part_out.at[w])


def _qkv0_body(x_ref, emb_ref, w_ref, h_ref, q_ref, kv_ref):
    xrow = x_ref[0]
    onehot_t = (lax.broadcasted_iota(jnp.int32, (128, ROWS), 0) == xrow
                ).astype(jnp.float32)
    h = lax.dot_general(onehot_t, emb_ref[...], (((0,), (0,)), ((), ())),
                        preferred_element_type=jnp.float32)
    qkv = jnp.dot(h, w_ref[...], preferred_element_type=jnp.float32)
    h_ref[...] = h
    q_ref[...] = (qkv[:, D:2 * D] * SCALE).astype(jnp.bfloat16)
    kv_ref[:, :D] = qkv[:, :D].astype(jnp.bfloat16)
    kv_ref[:, D:] = qkv[:, 2 * D:].astype(jnp.bfloat16)


def _qkv1_body(h_ref, w_ref, q_ref, kv_ref):
    qkv = jnp.dot(h_ref[...], w_ref[...], preferred_element_type=jnp.float32)
    q_ref[...] = (qkv[:, D:2 * D] * SCALE).astype(jnp.bfloat16)
    kv_ref[:, :D] = qkv[:, :D].astype(jnp.bfloat16)
    kv_ref[:, D:] = qkv[:, 2 * D:].astype(jnp.bfloat16)


def _post_body(h_ref, num_ref, den_ref, wo_ref, g1_ref, b1_ref, g2_ref,
               b2_ref, w1_ref, bf1_ref, w2_ref, bf2_ref, out_ref):
    rows = num_ref.shape[0]
    ih = lax.broadcasted_iota(jnp.int32, (16, D), 0)
    idd = lax.broadcasted_iota(jnp.int32, (16, D), 1)
    expand = (idd // HD == ih).astype(jnp.float32)
    den_rep = jnp.dot(den_ref[...], expand, preferred_element_type=jnp.float32)
    att = num_ref[...] / jnp.maximum(den_rep, 1e-16)
    att = jnp.dot(att, wo_ref[...], preferred_element_type=jnp.float32)
    h1 = h_ref[...] + att
    mu = h1.mean(-1, keepdims=True)
    var = ((h1 - mu) ** 2).mean(-1, keepdims=True)
    h1 = (h1 - mu) * lax.rsqrt(var + 1e-5) * g1_ref[...] + b1_ref[...]
    ff = jnp.maximum(
        jnp.dot(h1, w1_ref[...], preferred_element_type=jnp.float32)
        + bf1_ref[...], 0.0)
    ff = jnp.dot(ff, w2_ref[...], preferred_element_type=jnp.float32) \
        + bf2_ref[...]
    h2 = h1 + ff
    mu = h2.mean(-1, keepdims=True)
    var = ((h2 - mu) ** 2).mean(-1, keepdims=True)
    out_ref[...] = (h2 - mu) * lax.rsqrt(var + 1e-5) * g2_ref[...] \
        + b2_ref[...]


def _combine_body(part_ref, out_ref):
    acc = part_ref[0]
    for i in range(1, NW):
        acc = jnp.maximum(acc, part_ref[i])
    out_ref[...] = jnp.where(jnp.isfinite(acc), acc, 0.0)


ROWS = 400
GRID = N // ROWS


def _full(shape):
    return pl.BlockSpec(shape, lambda i: (0,) * len(shape))


def _rows(width):
    return pl.BlockSpec((ROWS, width), lambda i: (i, 0))


def _tc_qkv0(x3, emb, wcat):
    return pl.pallas_call(
        _qkv0_body,
        grid=(GRID,),
        in_specs=[
            pl.BlockSpec((1, 1, ROWS), lambda i: (i, 0, 0)),
            _full((128, D)),
            _full((D, 3 * D)),
        ],
        out_specs=[_rows(D), _rows(D), _rows(2 * D)],
        out_shape=[jax.ShapeDtypeStruct((N, D), jnp.float32),
                   jax.ShapeDtypeStruct((N, D), jnp.bfloat16),
                   jax.ShapeDtypeStruct((N, 2 * D), jnp.bfloat16)],
    )(x3, emb, wcat)


def _tc_qkv1(h, wcat):
    return pl.pallas_call(
        _qkv1_body,
        grid=(GRID,),
        in_specs=[_rows(D), _full((D, 3 * D))],
        out_specs=[_rows(D), _rows(2 * D)],
        out_shape=[jax.ShapeDtypeStruct((N, D), jnp.bfloat16),
                   jax.ShapeDtypeStruct((N, 2 * D), jnp.bfloat16)],
    )(h, wcat)


def _pack16(a):
    return lax.bitcast_convert_type(
        a.reshape(N, a.shape[1] // 2, 2), jnp.int32)


def _tc_post(h, num, den, wo, g1, b1, g2, b2, w1, bf1, w2, bf2):
    return pl.pallas_call(
        _post_body,
        grid=(GRID,),
        in_specs=[
            _rows(D), _rows(D), _rows(16), _full((D, D)),
            _full((1, D)), _full((1, D)), _full((1, D)), _full((1, D)),
            _full((D, FF)), _full((1, FF)), _full((FF, D)), _full((1, D)),
        ],
        out_specs=_rows(D),
        out_shape=jax.ShapeDtypeStruct((N, D), jnp.float32),
    )(h, num, den, wo, g1.reshape(1, D), b1.reshape(1, D),
      g2.reshape(1, D), b2.reshape(1, D), w1, bf1.reshape(1, FF), w2,
      bf2.reshape(1, D))


def _tc_combine(parts):
    return pl.pallas_call(
        _combine_body,
        out_shape=jax.ShapeDtypeStruct((B, D), jnp.float32),
    )(parts)


def _layer(h, src, dst_parts, wqk, wv, wo, g1, b1, g2, b2, w1, bf1, w2, bf2,
           x3=None, emb=None):
    dloc, gsrc = dst_parts
    wcat = jnp.concatenate([wqk, wv], axis=1)
    if x3 is not None:
        h, q, kv = _tc_qkv0(x3, emb, wcat)
    else:
        q, kv = _tc_qkv1(h, wcat)
    num, den = _edge_attention(_pack16(q), _pack16(kv), dloc, gsrc)
    return _tc_post(h, num[:N], den[:N], wo, g1, b1, g2, b2, w1, bf1, w2,
                    bf2)


def kernel(x, complete_edge_index, ptr, batch, emb,
           W_qk_0, W_v_0, W_o_0, ln1_g_0, ln1_b_0, ln2_g_0, ln2_b_0,
           W1_0, b1_0, W2_0, b2_0,
           W_qk_1, W_v_1, W_o_1, ln1_g_1, ln1_b_1, ln2_g_1, ln2_b_1,
           W1_1, b1_1, W2_1, b2_1):
    src = complete_edge_index[0].astype(jnp.int32)
    dst = complete_edge_index[1].astype(jnp.int32)
    parts = _edge_partition(dst, src)
    x3 = x.astype(jnp.int32).reshape(GRID, 1, ROWS)
    h = _layer(None, src, parts, W_qk_0, W_v_0, W_o_0, ln1_g_0, ln1_b_0,
               ln2_g_0, ln2_b_0, W1_0, b1_0, W2_0, b2_0, x3=x3, emb=emb)
    h = _layer(h, src, parts, W_qk_1, W_v_1, W_o_1, ln1_g_1, ln1_b_1,
               ln2_g_1, ln2_b_1, W1_1, b1_1, W2_1, b2_1)
    partials = _batch_max(h, batch.astype(jnp.int32))
    return _tc_combine(partials)


# speedup vs baseline: 1.7906x; 1.1091x over previous
import functools

import jax
import jax.numpy as jnp
from jax import lax
from jax.experimental import pallas as pl
from jax.experimental.pallas import tpu as pltpu
from jax.experimental.pallas import tpu_sc as plsc

N = 10000
E = 160000
D = 256
H = 8
HD = 32
FF = 512
B = 64
SCALE = HD ** -0.5

NC = 2
NS = 16
NW = NC * NS
NPH = 2
TROWS = 160
NL = NW * NPH
NOUT = NL * TROWS
ECHUNK = E // NS
LB = 1024
CAP = 4128
CE = 64
DW = D // 2

def _sc_mesh():
    return plsc.VectorSubcoreMesh(core_axis_name="c", subcore_axis_name="s",
                                  num_cores=NC, num_subcores=NS)


@functools.cache
def _edge_partition_kernel():
    return pl.kernel(
        _edge_partition_body,
        out_type=(
            jax.ShapeDtypeStruct((NL * CAP,), jnp.int32),
            jax.ShapeDtypeStruct((NL * CAP,), jnp.int32),
        ),
        mesh=_sc_mesh(),
        scratch_types=[
            pltpu.VMEM((ECHUNK,), jnp.int32),
            pltpu.VMEM((ECHUNK,), jnp.int32),
            pltpu.VMEM((CAP,), jnp.int32),
            pltpu.VMEM((CAP,), jnp.int32),
            pltpu.VMEM((CAP,), jnp.int32),
            pltpu.VMEM((CAP,), jnp.int32),
        ],
        compiler_params=pltpu.CompilerParams(needs_layout_passes=False),
    )


def _edge_partition(dst, src):
    return _edge_partition_kernel()(dst, src)


def _edge_partition_body(dst_hbm, src_hbm, dloc_out, gsrc_out,
                         dbuf, sbuf, od0, os0, od1, os1):
    c = lax.axis_index("c")
    s = lax.axis_index("s")
    w = c * NS + s

    pad_d = jnp.full((16,), TROWS, dtype=jnp.int32)
    pad_s = jnp.zeros((16,), dtype=jnp.int32)

    def prefill(i, _):
        od0[pl.ds(i * 16, 16)] = pad_d
        os0[pl.ds(i * 16, 16)] = pad_s
        od1[pl.ds(i * 16, 16)] = pad_d
        os1[pl.ds(i * 16, 16)] = pad_s
        return 0

    lax.fori_loop(0, CAP // 16, prefill, 0)

    lo0 = w * TROWS
    lo1 = NW * TROWS + w * TROWS

    def outer(ch, curs):
        pltpu.sync_copy(dst_hbm.at[pl.ds(ch * ECHUNK, ECHUNK)], dbuf)
        pltpu.sync_copy(src_hbm.at[pl.ds(ch * ECHUNK, ECHUNK)], sbuf)

        def scan(i, curs):
            cur0, cur1 = curs
            dv = dbuf[pl.ds(i * 16, 16)]
            sv = sbuf[pl.ds(i * 16, 16)]
            m0 = (dv >= lo0) & (dv < lo0 + TROWS)
            cs0 = plsc.cumsum(m0.astype(jnp.int32))
            pos0 = jnp.minimum(cur0 + cs0 - 1, CAP - 17)
            plsc.store_scatter(od0, [pos0], dv - lo0, mask=m0)
            plsc.store_scatter(os0, [pos0], sv, mask=m0)
            m1 = (dv >= lo1) & (dv < lo1 + TROWS)
            cs1 = plsc.cumsum(m1.astype(jnp.int32))
            pos1 = jnp.minimum(cur1 + cs1 - 1, CAP - 17)
            plsc.store_scatter(od1, [pos1], dv - lo1, mask=m1)
            plsc.store_scatter(os1, [pos1], sv, mask=m1)
            return (cur0 + cs0[15], cur1 + cs1[15])

        return lax.fori_loop(0, ECHUNK // 16, scan, curs)

    t0, t1 = lax.fori_loop(0, NS, outer, (jnp.int32(0), jnp.int32(0)))
    t0 = jnp.minimum(t0, CAP - 32)
    t1 = jnp.minimum(t1, CAP - 32)

    od0[pl.ds(CAP - 16, 16)] = jnp.full((16,), t0, dtype=jnp.int32)
    od1[pl.ds(CAP - 16, 16)] = jnp.full((16,), t1, dtype=jnp.int32)
    pltpu.sync_copy(od0, dloc_out.at[pl.ds(w * CAP, CAP)])
    pltpu.sync_copy(os0, gsrc_out.at[pl.ds(w * CAP, CAP)])
    pltpu.sync_copy(od1, dloc_out.at[pl.ds((NW + w) * CAP, CAP)])
    pltpu.sync_copy(os1, gsrc_out.at[pl.ds((NW + w) * CAP, CAP)])


@functools.cache
def _edge_attention_kernel():
    return pl.kernel(
        _edge_attention_body,
        out_type=(
            jax.ShapeDtypeStruct((NOUT, D), jnp.float32),
            jax.ShapeDtypeStruct((NOUT, 16), jnp.float32),
        ),
        mesh=_sc_mesh(),
        scratch_types=[
            [pltpu.VMEM((CE,), jnp.int32)] * 2,
            [pltpu.VMEM((CE,), jnp.int32)] * 2,
            [pltpu.VMEM((CE,), jnp.int32)] * 2,
            [pltpu.VMEM((CE,), jnp.int32)] * 2,
            [pltpu.VMEM((CE,), jnp.int32)] * 2,
            [pltpu.VMEM((CE, DW), jnp.int32)] * 2,
            [pltpu.VMEM((CE, DW), jnp.int32)] * 2,
            [pltpu.VMEM((CE, DW), jnp.int32)] * 2,
            pltpu.VMEM((CE, 16), jnp.float32),
            pltpu.VMEM((TROWS + 8, D), jnp.float32),
            pltpu.VMEM((TROWS + 8, 16), jnp.float32),
            [pltpu.SemaphoreType.DMA] * 2,
            [pltpu.SemaphoreType.DMA] * 2,
        ],
        compiler_params=pltpu.CompilerParams(needs_layout_passes=False),
    )


def _edge_attention(q, k, v, dloc, gsrc):
    return _edge_attention_kernel()(q, k, v, dloc, gsrc)


def _edge_attention_body(q_hbm, k_hbm, v_hbm, dloc_hbm, gsrc_hbm,
                         num_out, den_out,
                         dchunk, schunk, didx, qidx, sidx, qrows, krows,
                         vrows, exbuf, nacc, dacc, semg, seml):
    c = lax.axis_index("c")
    s = lax.axis_index("s")
    w = c * NS + s
    zv = jnp.zeros((16,), dtype=jnp.float32)
    iota16 = lax.iota(jnp.int32, 16)

    def lbase(j):
        return jnp.minimum(j * CE, CAP - CE)

    def phase(p, _):
        lrow = p * NW + w

        pltpu.sync_copy(dloc_hbm.at[pl.ds(lrow * CAP + CAP - 16, 16)],
                        qidx[0].at[pl.ds(0, 16)])
        n_e = qidx[0][pl.ds(0, 16)][0]

        def znum(i, _):
            nacc[i // 16, pl.ds((i % 16) * 16, 16)] = zv
            return 0

        def zden(i, _):
            dacc[i, pl.ds(0, 16)] = zv
            return 0

        lax.fori_loop(0, (TROWS + 8) * 16, znum, 0)
        lax.fori_loop(0, TROWS + 8, zden, 0)

        qoff = lrow * TROWS

        def fire_lists(j, b):
            bj = lrow * CAP + lbase(j)
            pltpu.async_copy(dloc_hbm.at[pl.ds(bj, CE)], dchunk[b], seml[b])
            pltpu.async_copy(gsrc_hbm.at[pl.ds(bj, CE)], schunk[b], seml[b])

        def wait_lists(j, b):
            bj = lrow * CAP + lbase(j)
            pltpu.make_async_copy(dloc_hbm.at[pl.ds(bj, CE)], dchunk[b],
                                  seml[b]).wait()
            pltpu.make_async_copy(gsrc_hbm.at[pl.ds(bj, CE)], schunk[b],
                                  seml[b]).wait()

        def build_idx(b):
            for t in range(CE // 16):
                dv = jnp.minimum(dchunk[b][pl.ds(t * 16, 16)], TROWS)
                sv = schunk[b][pl.ds(t * 16, 16)]
                didx[b][pl.ds(t * 16, 16)] = dv
                qidx[b][pl.ds(t * 16, 16)] = jnp.minimum(dv + qoff, N - 1)
                sidx[b][pl.ds(t * 16, 16)] = sv

        def fire_gathers(b):
            pltpu.async_copy(q_hbm.at[qidx[b]], qrows[b], semg[b])
            pltpu.async_copy(k_hbm.at[sidx[b]], krows[b], semg[b])
            pltpu.async_copy(v_hbm.at[sidx[b]], vrows[b], semg[b])

        def wait_gathers(b):
            pltpu.make_async_copy(q_hbm.at[qidx[b]], qrows[b], semg[b]).wait()
            pltpu.make_async_copy(k_hbm.at[sidx[b]], krows[b], semg[b]).wait()
            pltpu.make_async_copy(v_hbm.at[sidx[b]], vrows[b], semg[b]).wait()

        mhi = jnp.int32(-65536)

        def unlo(wv):
            return lax.bitcast_convert_type(wv << 16, jnp.float32)

        def unhi(wv):
            return lax.bitcast_convert_type(wv & mhi, jnp.float32)

        def compute(i, b):
            def edge(e, _):
                ex = jnp.zeros((16,), dtype=jnp.float32)
                for h in range(H):
                    qw = qrows[b][e, pl.ds(h * 16, 16)]
                    kw = krows[b][e, pl.ds(h * 16, 16)]
                    pr = unlo(qw) * unlo(kw) + unhi(qw) * unhi(kw)
                    sh = jnp.sum(pr)
                    ex = jnp.where(iota16 == h,
                                   jnp.full((16,), sh, dtype=jnp.float32),
                                   ex)
                exbuf[e, pl.ds(0, 16)] = jnp.exp(ex)
                return 0

            lax.fori_loop(0, CE, edge, 0)

            def wgroup(t, _):
                dloc16 = didx[b][pl.ds(t * 16, 16)]
                for r in range(16):
                    e = t * 16 + r
                    ev = exbuf[e, pl.ds(0, 16)]
                    rowv = jnp.full((16,), dloc16[r], dtype=jnp.int32)
                    plsc.addupdate_scatter(dacc, [rowv, iota16], ev)
                    for h in range(H):
                        a = jnp.full((16,), ev[h], dtype=jnp.float32)
                        vw = vrows[b][e, pl.ds(h * 16, 16)]
                        col = h * HD + 2 * iota16
                        plsc.addupdate_scatter(
                            nacc, [rowv, col], unlo(vw) * a)
                        plsc.addupdate_scatter(
                            nacc, [rowv, col + 1], unhi(vw) * a)
                return 0

            lax.fori_loop(0, CE // 16, wgroup, 0)

        fire_lists(0, 0)
        wait_lists(0, 0)
        build_idx(0)
        fire_gathers(0)
        fire_lists(1, 1)

        nchp = ((n_e + (CE - 1)) // CE + 1) // 2

        def pair(i2, _):
            for b in range(2):
                i = 2 * i2 + b
                bn = 1 - b
                wait_gathers(b)
                wait_lists(i + 1, bn)
                build_idx(bn)
                fire_gathers(bn)
                fire_lists(i + 2, b)
                compute(i, b)
            return 0

        lax.fori_loop(0, nchp, pair, 0)

        wait_gathers(0)
        wait_lists(2 * nchp + 1, 1)

        pltpu.sync_copy(nacc.at[pl.ds(0, TROWS)],
                        num_out.at[pl.ds(qoff, TROWS)])
        pltpu.sync_copy(dacc.at[pl.ds(0, TROWS)],
                        den_out.at[pl.ds(qoff, TROWS)])
        return 0

    lax.fori_loop(0, NPH, phase, 0)


SROWS = 320


@functools.cache
def _batch_max_kernel():
    return pl.kernel(
        _batch_max_body,
        out_type=jax.ShapeDtypeStruct((NW, B, D), jnp.float32),
        mesh=_sc_mesh(),
        scratch_types=[
            pltpu.VMEM((16, D), jnp.float32),
            pltpu.VMEM((16,), jnp.int32),
            pltpu.VMEM((B, D), jnp.float32),
        ],
        compiler_params=pltpu.CompilerParams(needs_layout_passes=False),
    )


def _batch_max(h, bid):
    return _batch_max_kernel()(h, bid)


def _batch_max_body(h_hbm, bid_hbm, part_out, rowbuf, bbuf, acc):
    c = lax.axis_index("c")
    s = lax.axis_index("s")
    w = c * NS + s
    n0 = w * SROWS
    nr = jnp.clip(N - n0, 0, SROWS)

    ninf = jnp.full((16,), -jnp.inf, dtype=jnp.float32)

    def zacc(i, _):
        acc[i // 16, pl.ds((i % 16) * 16, 16)] = ninf
        return 0

    lax.fori_loop(0, B * 16, zacc, 0)

    def chunkfn(ci, _):
        base = n0 + ci * 16
        pltpu.sync_copy(h_hbm.at[pl.ds(base, 16)], rowbuf)
        pltpu.sync_copy(bid_hbm.at[pl.ds(base, 16)], bbuf)

        bv = bbuf[pl.ds(0, 16)]
        for r in range(16):
            bid = bv[r]
            for j in range(D // 16):
                cur = acc[bid, pl.ds(j * 16, 16)]
                acc[bid, pl.ds(j * 16, 16)] = jnp.maximum(
                    cur, rowbuf[r, pl.ds(j * 16, 16)])
        return 0

    lax.fori_loop(0, nr // 16, chunkfn, 0)
    pltpu.sync_copy(acc, part_out.at[w])


def _qkv0_body(x_ref, emb_ref, w_ref, h_ref, q_ref, k_ref, v_ref):
    xrow = x_ref[0]
    onehot_t = (lax.broadcasted_iota(jnp.int32, (128, ROWS), 0) == xrow
                ).astype(jnp.float32)
    h = lax.dot_general(onehot_t, emb_ref[...], (((0,), (0,)), ((), ())),
                        preferred_element_type=jnp.float32)
    qkv = jnp.dot(h, w_ref[...], preferred_element_type=jnp.float32)
    h_ref[...] = h
    q_ref[...] = (qkv[:, D:2 * D] * SCALE).astype(jnp.bfloat16)
    k_ref[...] = qkv[:, :D].astype(jnp.bfloat16)
    v_ref[...] = qkv[:, 2 * D:].astype(jnp.bfloat16)


def _qkv1_body(h_ref, w_ref, q_ref, k_ref, v_ref):
    qkv = jnp.dot(h_ref[...], w_ref[...], preferred_element_type=jnp.float32)
    q_ref[...] = (qkv[:, D:2 * D] * SCALE).astype(jnp.bfloat16)
    k_ref[...] = qkv[:, :D].astype(jnp.bfloat16)
    v_ref[...] = qkv[:, 2 * D:].astype(jnp.bfloat16)


def _post_body(h_ref, num_ref, den_ref, wo_ref, g1_ref, b1_ref, g2_ref,
               b2_ref, w1_ref, bf1_ref, w2_ref, bf2_ref, out_ref):
    rows = num_ref.shape[0]
    ih = lax.broadcasted_iota(jnp.int32, (16, D), 0)
    idd = lax.broadcasted_iota(jnp.int32, (16, D), 1)
    expand = (idd // HD == ih).astype(jnp.float32)
    den_rep = jnp.dot(den_ref[...], expand, preferred_element_type=jnp.float32)
    att = num_ref[...] / jnp.maximum(den_rep, 1e-16)
    att = jnp.dot(att, wo_ref[...], preferred_element_type=jnp.float32)
    h1 = h_ref[...] + att
    mu = h1.mean(-1, keepdims=True)
    var = ((h1 - mu) ** 2).mean(-1, keepdims=True)
    h1 = (h1 - mu) * lax.rsqrt(var + 1e-5) * g1_ref[...] + b1_ref[...]
    ff = jnp.maximum(
        jnp.dot(h1, w1_ref[...], preferred_element_type=jnp.float32)
        + bf1_ref[...], 0.0)
    ff = jnp.dot(ff, w2_ref[...], preferred_element_type=jnp.float32) \
        + bf2_ref[...]
    h2 = h1 + ff
    mu = h2.mean(-1, keepdims=True)
    var = ((h2 - mu) ** 2).mean(-1, keepdims=True)
    out_ref[...] = (h2 - mu) * lax.rsqrt(var + 1e-5) * g2_ref[...] \
        + b2_ref[...]


def _combine_body(part_ref, out_ref):
    acc = part_ref[0]
    for i in range(1, NW):
        acc = jnp.maximum(acc, part_ref[i])
    out_ref[...] = jnp.where(jnp.isfinite(acc), acc, 0.0)


ROWS = 400
GRID = N // ROWS


def _full(shape):
    return pl.BlockSpec(shape, lambda i: (0,) * len(shape))


def _rows(width):
    return pl.BlockSpec((ROWS, width), lambda i: (i, 0))


def _tc_qkv0(x3, emb, wcat):
    return pl.pallas_call(
        _qkv0_body,
        grid=(GRID,),
        in_specs=[
            pl.BlockSpec((1, 1, ROWS), lambda i: (i, 0, 0)),
            _full((128, D)),
            _full((D, 3 * D)),
        ],
        out_specs=[_rows(D), _rows(D), _rows(D), _rows(D)],
        out_shape=[jax.ShapeDtypeStruct((N, D), jnp.float32)] +
        [jax.ShapeDtypeStruct((N, D), jnp.bfloat16)] * 3,
    )(x3, emb, wcat)


def _tc_qkv1(h, wcat):
    return pl.pallas_call(
        _qkv1_body,
        grid=(GRID,),
        in_specs=[_rows(D), _full((D, 3 * D))],
        out_specs=[_rows(D), _rows(D), _rows(D)],
        out_shape=[jax.ShapeDtypeStruct((N, D), jnp.bfloat16)] * 3,
    )(h, wcat)


def _pack16(a):
    return lax.bitcast_convert_type(
        a.reshape(N, a.shape[1] // 2, 2), jnp.int32)


def _tc_post(h, num, den, wo, g1, b1, g2, b2, w1, bf1, w2, bf2):
    return pl.pallas_call(
        _post_body,
        grid=(GRID,),
        in_specs=[
            _rows(D), _rows(D), _rows(16), _full((D, D)),
            _full((1, D)), _full((1, D)), _full((1, D)), _full((1, D)),
            _full((D, FF)), _full((1, FF)), _full((FF, D)), _full((1, D)),
        ],
        out_specs=_rows(D),
        out_shape=jax.ShapeDtypeStruct((N, D), jnp.float32),
    )(h, num, den, wo, g1.reshape(1, D), b1.reshape(1, D),
      g2.reshape(1, D), b2.reshape(1, D), w1, bf1.reshape(1, FF), w2,
      bf2.reshape(1, D))


def _tc_combine(parts):
    return pl.pallas_call(
        _combine_body,
        out_shape=jax.ShapeDtypeStruct((B, D), jnp.float32),
    )(parts)


def _layer(h, src, dst_parts, wqk, wv, wo, g1, b1, g2, b2, w1, bf1, w2, bf2,
           x3=None, emb=None):
    dloc, gsrc = dst_parts
    wcat = jnp.concatenate([wqk, wv], axis=1)
    if x3 is not None:
        h, q, k, v = _tc_qkv0(x3, emb, wcat)
    else:
        q, k, v = _tc_qkv1(h, wcat)
    num, den = _edge_attention(_pack16(q), _pack16(k), _pack16(v),
                               dloc, gsrc)
    return _tc_post(h, num[:N], den[:N], wo, g1, b1, g2, b2, w1, bf1, w2,
                    bf2)


def kernel(x, complete_edge_index, ptr, batch, emb,
           W_qk_0, W_v_0, W_o_0, ln1_g_0, ln1_b_0, ln2_g_0, ln2_b_0,
           W1_0, b1_0, W2_0, b2_0,
           W_qk_1, W_v_1, W_o_1, ln1_g_1, ln1_b_1, ln2_g_1, ln2_b_1,
           W1_1, b1_1, W2_1, b2_1):
    src = complete_edge_index[0].astype(jnp.int32)
    dst = complete_edge_index[1].astype(jnp.int32)
    parts = _edge_partition(dst, src)
    x3 = x.astype(jnp.int32).reshape(GRID, 1, ROWS)
    h = _layer(None, src, parts, W_qk_0, W_v_0, W_o_0, ln1_g_0, ln1_b_0,
               ln2_g_0, ln2_b_0, W1_0, b1_0, W2_0, b2_0, x3=x3, emb=emb)
    h = _layer(h, src, parts, W_qk_1, W_v_1, W_o_1, ln1_g_1, ln1_b_1,
               ln2_g_1, ln2_b_1, W1_1, b1_1, W2_1, b2_1)
    partials = _batch_max(h, batch.astype(jnp.int32))
    return _tc_combine(partials)


# speedup vs baseline: 1.7911x; 1.0003x over previous
import functools

import jax
import jax.numpy as jnp
from jax import lax
from jax.experimental import pallas as pl
from jax.experimental.pallas import tpu as pltpu
from jax.experimental.pallas import tpu_sc as plsc

N = 10000
E = 160000
D = 256
H = 8
HD = 32
FF = 512
B = 64
SCALE = HD ** -0.5

NC = 2
NS = 16
NW = NC * NS
NPH = 2
TROWS = 160
NL = NW * NPH
NOUT = NL * TROWS
ECHUNK = E // NS
LB = 1024
CAP = 4128
CE = 64
DW = D // 2

def _sc_mesh():
    return plsc.VectorSubcoreMesh(core_axis_name="c", subcore_axis_name="s",
                                  num_cores=NC, num_subcores=NS)


@functools.cache
def _edge_partition_kernel():
    return pl.kernel(
        _edge_partition_body,
        out_type=(
            jax.ShapeDtypeStruct((NL * CAP,), jnp.int32),
            jax.ShapeDtypeStruct((NL * CAP,), jnp.int32),
        ),
        mesh=_sc_mesh(),
        scratch_types=[
            pltpu.VMEM((ECHUNK,), jnp.int32),
            pltpu.VMEM((ECHUNK,), jnp.int32),
            pltpu.VMEM((CAP,), jnp.int32),
            pltpu.VMEM((CAP,), jnp.int32),
            pltpu.VMEM((CAP,), jnp.int32),
            pltpu.VMEM((CAP,), jnp.int32),
        ],
        compiler_params=pltpu.CompilerParams(needs_layout_passes=False),
    )


def _edge_partition(dst, src):
    return _edge_partition_kernel()(dst, src)


def _edge_partition_body(dst_hbm, src_hbm, dloc_out, gsrc_out,
                         dbuf, sbuf, od0, os0, od1, os1):
    c = lax.axis_index("c")
    s = lax.axis_index("s")
    w = c * NS + s

    pad_d = jnp.full((16,), TROWS, dtype=jnp.int32)
    pad_s = jnp.zeros((16,), dtype=jnp.int32)

    def prefill(i, _):
        od0[pl.ds(i * 16, 16)] = pad_d
        os0[pl.ds(i * 16, 16)] = pad_s
        od1[pl.ds(i * 16, 16)] = pad_d
        os1[pl.ds(i * 16, 16)] = pad_s
        return 0

    lax.fori_loop(0, CAP // 16, prefill, 0)

    lo0 = w * TROWS
    lo1 = NW * TROWS + w * TROWS

    def outer(ch, curs):
        pltpu.sync_copy(dst_hbm.at[pl.ds(ch * ECHUNK, ECHUNK)], dbuf)
        pltpu.sync_copy(src_hbm.at[pl.ds(ch * ECHUNK, ECHUNK)], sbuf)

        def scan(i, curs):
            cur0, cur1 = curs
            dv = dbuf[pl.ds(i * 16, 16)]
            sv = sbuf[pl.ds(i * 16, 16)]
            m0 = (dv >= lo0) & (dv < lo0 + TROWS)
            cs0 = plsc.cumsum(m0.astype(jnp.int32))
            pos0 = jnp.minimum(cur0 + cs0 - 1, CAP - 17)
            plsc.store_scatter(od0, [pos0], dv - lo0, mask=m0)
            plsc.store_scatter(os0, [pos0], sv, mask=m0)
            m1 = (dv >= lo1) & (dv < lo1 + TROWS)
            cs1 = plsc.cumsum(m1.astype(jnp.int32))
            pos1 = jnp.minimum(cur1 + cs1 - 1, CAP - 17)
            plsc.store_scatter(od1, [pos1], dv - lo1, mask=m1)
            plsc.store_scatter(os1, [pos1], sv, mask=m1)
            return (cur0 + cs0[15], cur1 + cs1[15])

        return lax.fori_loop(0, ECHUNK // 16, scan, curs)

    t0, t1 = lax.fori_loop(0, NS, outer, (jnp.int32(0), jnp.int32(0)))
    t0 = jnp.minimum(t0, CAP - 32)
    t1 = jnp.minimum(t1, CAP - 32)

    od0[pl.ds(CAP - 16, 16)] = jnp.full((16,), t0, dtype=jnp.int32)
    od1[pl.ds(CAP - 16, 16)] = jnp.full((16,), t1, dtype=jnp.int32)
    pltpu.sync_copy(od0, dloc_out.at[pl.ds(w * CAP, CAP)])
    pltpu.sync_copy(os0, gsrc_out.at[pl.ds(w * CAP, CAP)])
    pltpu.sync_copy(od1, dloc_out.at[pl.ds((NW + w) * CAP, CAP)])
    pltpu.sync_copy(os1, gsrc_out.at[pl.ds((NW + w) * CAP, CAP)])


@functools.cache
def _edge_attention_kernel():
    return pl.kernel(
        _edge_attention_body,
        out_type=(
            jax.ShapeDtypeStruct((NOUT, D), jnp.float32),
            jax.ShapeDtypeStruct((NOUT, 16), jnp.float32),
        ),
        mesh=_sc_mesh(),
        scratch_types=[
            [pltpu.VMEM((CE,), jnp.int32)] * 2,
            [pltpu.VMEM((CE,), jnp.int32)] * 2,
            [pltpu.VMEM((CE,), jnp.int32)] * 2,
            [pltpu.VMEM((CE,), jnp.int32)] * 2,
            [pltpu.VMEM((CE,), jnp.int32)] * 2,
            [pltpu.VMEM((CE, DW), jnp.int32)] * 2,
            [pltpu.VMEM((CE, DW), jnp.int32)] * 2,
            [pltpu.VMEM((CE, DW), jnp.int32)] * 2,
            pltpu.VMEM((CE, 16), jnp.float32),
            pltpu.VMEM((TROWS + 8, D), jnp.float32),
            pltpu.VMEM((TROWS + 8, 16), jnp.float32),
            [pltpu.SemaphoreType.DMA] * 2,
            [pltpu.SemaphoreType.DMA] * 2,
        ],
        compiler_params=pltpu.CompilerParams(needs_layout_passes=False),
    )


def _edge_attention(q, k, v, dloc, gsrc):
    return _edge_attention_kernel()(q, k, v, dloc, gsrc)


def _edge_attention_body(q_hbm, k_hbm, v_hbm, dloc_hbm, gsrc_hbm,
                         num_out, den_out,
                         dchunk, schunk, didx, qidx, sidx, qrows, krows,
                         vrows, exbuf, nacc, dacc, semg, seml):
    c = lax.axis_index("c")
    s = lax.axis_index("s")
    w = c * NS + s
    zv = jnp.zeros((16,), dtype=jnp.float32)
    iota16 = lax.iota(jnp.int32, 16)

    def lbase(j):
        return jnp.minimum(j * CE, CAP - CE)

    def phase(p, _):
        lrow = p * NW + w

        pltpu.sync_copy(dloc_hbm.at[pl.ds(lrow * CAP + CAP - 16, 16)],
                        qidx[0].at[pl.ds(0, 16)])
        n_e = qidx[0][pl.ds(0, 16)][0]

        def znum(i, _):
            nacc[i // 16, pl.ds((i % 16) * 16, 16)] = zv
            return 0

        def zden(i, _):
            dacc[i, pl.ds(0, 16)] = zv
            return 0

        lax.fori_loop(0, (TROWS + 8) * 16, znum, 0)
        lax.fori_loop(0, TROWS + 8, zden, 0)

        qoff = lrow * TROWS

        def fire_lists(j, b):
            bj = lrow * CAP + lbase(j)
            pltpu.async_copy(dloc_hbm.at[pl.ds(bj, CE)], dchunk[b], seml[b])
            pltpu.async_copy(gsrc_hbm.at[pl.ds(bj, CE)], schunk[b], seml[b])

        def wait_lists(j, b):
            bj = lrow * CAP + lbase(j)
            pltpu.make_async_copy(dloc_hbm.at[pl.ds(bj, CE)], dchunk[b],
                                  seml[b]).wait()
            pltpu.make_async_copy(gsrc_hbm.at[pl.ds(bj, CE)], schunk[b],
                                  seml[b]).wait()

        def build_idx(b):
            for t in range(CE // 16):
                dv = jnp.minimum(dchunk[b][pl.ds(t * 16, 16)], TROWS)
                sv = schunk[b][pl.ds(t * 16, 16)]
                didx[b][pl.ds(t * 16, 16)] = dv
                qidx[b][pl.ds(t * 16, 16)] = jnp.minimum(dv + qoff, N - 1)
                sidx[b][pl.ds(t * 16, 16)] = sv

        GP = 4
        GS = CE // GP

        def fire_gathers(b):
            for g in range(GP):
                sl = pl.ds(g * GS, GS)
                pltpu.async_copy(q_hbm.at[qidx[b].at[sl]], qrows[b].at[sl],
                                 semg[b])
                pltpu.async_copy(k_hbm.at[sidx[b].at[sl]], krows[b].at[sl],
                                 semg[b])
                pltpu.async_copy(v_hbm.at[sidx[b].at[sl]], vrows[b].at[sl],
                                 semg[b])

        def wait_gathers(b):
            for g in range(GP):
                sl = pl.ds(g * GS, GS)
                pltpu.make_async_copy(q_hbm.at[qidx[b].at[sl]],
                                      qrows[b].at[sl], semg[b]).wait()
                pltpu.make_async_copy(k_hbm.at[sidx[b].at[sl]],
                                      krows[b].at[sl], semg[b]).wait()
                pltpu.make_async_copy(v_hbm.at[sidx[b].at[sl]],
                                      vrows[b].at[sl], semg[b]).wait()

        mhi = jnp.int32(-65536)

        def unlo(wv):
            return lax.bitcast_convert_type(wv << 16, jnp.float32)

        def unhi(wv):
            return lax.bitcast_convert_type(wv & mhi, jnp.float32)

        def compute(i, b):
            def edge(e, _):
                ex = jnp.zeros((16,), dtype=jnp.float32)
                for h in range(H):
                    qw = qrows[b][e, pl.ds(h * 16, 16)]
                    kw = krows[b][e, pl.ds(h * 16, 16)]
                    pr = unlo(qw) * unlo(kw) + unhi(qw) * unhi(kw)
                    sh = jnp.sum(pr)
                    ex = jnp.where(iota16 == h,
                                   jnp.full((16,), sh, dtype=jnp.float32),
                                   ex)
                exbuf[e, pl.ds(0, 16)] = jnp.exp(ex)
                return 0

            lax.fori_loop(0, CE, edge, 0)

            def wgroup(t, _):
                dloc16 = didx[b][pl.ds(t * 16, 16)]
                for r in range(16):
                    e = t * 16 + r
                    ev = exbuf[e, pl.ds(0, 16)]
                    rowv = jnp.full((16,), dloc16[r], dtype=jnp.int32)
                    plsc.addupdate_scatter(dacc, [rowv, iota16], ev)
                    for h in range(H):
                        a = jnp.full((16,), ev[h], dtype=jnp.float32)
                        vw = vrows[b][e, pl.ds(h * 16, 16)]
                        col = h * HD + 2 * iota16
                        plsc.addupdate_scatter(
                            nacc, [rowv, col], unlo(vw) * a)
                        plsc.addupdate_scatter(
                            nacc, [rowv, col + 1], unhi(vw) * a)
                return 0

            lax.fori_loop(0, CE // 16, wgroup, 0)

        fire_lists(0, 0)
        wait_lists(0, 0)
        build_idx(0)
        fire_gathers(0)
        fire_lists(1, 1)

        nchp = ((n_e + (CE - 1)) // CE + 1) // 2

        def pair(i2, _):
            for b in range(2):
                i = 2 * i2 + b
                bn = 1 - b
                wait_gathers(b)
                wait_lists(i + 1, bn)
                build_idx(bn)
                fire_gathers(bn)
                fire_lists(i + 2, b)
                compute(i, b)
            return 0

        lax.fori_loop(0, nchp, pair, 0)

        wait_gathers(0)
        wait_lists(2 * nchp + 1, 1)

        pltpu.sync_copy(nacc.at[pl.ds(0, TROWS)],
                        num_out.at[pl.ds(qoff, TROWS)])
        pltpu.sync_copy(dacc.at[pl.ds(0, TROWS)],
                        den_out.at[pl.ds(qoff, TROWS)])
        return 0

    lax.fori_loop(0, NPH, phase, 0)


SROWS = 320


@functools.cache
def _batch_max_kernel():
    return pl.kernel(
        _batch_max_body,
        out_type=jax.ShapeDtypeStruct((NW, B, D), jnp.float32),
        mesh=_sc_mesh(),
        scratch_types=[
            pltpu.VMEM((16, D), jnp.float32),
            pltpu.VMEM((16,), jnp.int32),
            pltpu.VMEM((B, D), jnp.float32),
        ],
        compiler_params=pltpu.CompilerParams(needs_layout_passes=False),
    )


def _batch_max(h, bid):
    return _batch_max_kernel()(h, bid)


def _batch_max_body(h_hbm, bid_hbm, part_out, rowbuf, bbuf, acc):
    c = lax.axis_index("c")
    s = lax.axis_index("s")
    w = c * NS + s
    n0 = w * SROWS
    nr = jnp.clip(N - n0, 0, SROWS)

    ninf = jnp.full((16,), -jnp.inf, dtype=jnp.float32)

    def zacc(i, _):
        acc[i // 16, pl.ds((i % 16) * 16, 16)] = ninf
        return 0

    lax.fori_loop(0, B * 16, zacc, 0)

    def chunkfn(ci, _):
        base = n0 + ci * 16
        pltpu.sync_copy(h_hbm.at[pl.ds(base, 16)], rowbuf)
        pltpu.sync_copy(bid_hbm.at[pl.ds(base, 16)], bbuf)

        bv = bbuf[pl.ds(0, 16)]
        for r in range(16):
            bid = bv[r]
            for j in range(D // 16):
                cur = acc[bid, pl.ds(j * 16, 16)]
                acc[bid, pl.ds(j * 16, 16)] = jnp.maximum(
                    cur, rowbuf[r, pl.ds(j * 16, 16)])
        return 0

    lax.fori_loop(0, nr // 16, chunkfn, 0)
    pltpu.sync_copy(acc, part_out.at[w])


def _qkv0_body(x_ref, emb_ref, w_ref, h_ref, q_ref, k_ref, v_ref):
    xrow = x_ref[0]
    onehot_t = (lax.broadcasted_iota(jnp.int32, (128, ROWS), 0) == xrow
                ).astype(jnp.float32)
    h = lax.dot_general(onehot_t, emb_ref[...], (((0,), (0,)), ((), ())),
                        preferred_element_type=jnp.float32)
    qkv = jnp.dot(h, w_ref[...], preferred_element_type=jnp.float32)
    h_ref[...] = h
    q_ref[...] = (qkv[:, D:2 * D] * SCALE).astype(jnp.bfloat16)
    k_ref[...] = qkv[:, :D].astype(jnp.bfloat16)
    v_ref[...] = qkv[:, 2 * D:].astype(jnp.bfloat16)


def _qkv1_body(h_ref, w_ref, q_ref, k_ref, v_ref):
    qkv = jnp.dot(h_ref[...], w_ref[...], preferred_element_type=jnp.float32)
    q_ref[...] = (qkv[:, D:2 * D] * SCALE).astype(jnp.bfloat16)
    k_ref[...] = qkv[:, :D].astype(jnp.bfloat16)
    v_ref[...] = qkv[:, 2 * D:].astype(jnp.bfloat16)


def _post_body(h_ref, num_ref, den_ref, wo_ref, g1_ref, b1_ref, g2_ref,
               b2_ref, w1_ref, bf1_ref, w2_ref, bf2_ref, out_ref):
    rows = num_ref.shape[0]
    ih = lax.broadcasted_iota(jnp.int32, (16, D), 0)
    idd = lax.broadcasted_iota(jnp.int32, (16, D), 1)
    expand = (idd // HD == ih).astype(jnp.float32)
    den_rep = jnp.dot(den_ref[...], expand, preferred_element_type=jnp.float32)
    att = num_ref[...] / jnp.maximum(den_rep, 1e-16)
    att = jnp.dot(att, wo_ref[...], preferred_element_type=jnp.float32)
    h1 = h_ref[...] + att
    mu = h1.mean(-1, keepdims=True)
    var = ((h1 - mu) ** 2).mean(-1, keepdims=True)
    h1 = (h1 - mu) * lax.rsqrt(var + 1e-5) * g1_ref[...] + b1_ref[...]
    ff = jnp.maximum(
        jnp.dot(h1, w1_ref[...], preferred_element_type=jnp.float32)
        + bf1_ref[...], 0.0)
    ff = jnp.dot(ff, w2_ref[...], preferred_element_type=jnp.float32) \
        + bf2_ref[...]
    h2 = h1 + ff
    mu = h2.mean(-1, keepdims=True)
    var = ((h2 - mu) ** 2).mean(-1, keepdims=True)
    out_ref[...] = (h2 - mu) * lax.rsqrt(var + 1e-5) * g2_ref[...] \
        + b2_ref[...]


def _combine_body(part_ref, out_ref):
    acc = part_ref[0]
    for i in range(1, NW):
        acc = jnp.maximum(acc, part_ref[i])
    out_ref[...] = jnp.where(jnp.isfinite(acc), acc, 0.0)


ROWS = 400
GRID = N // ROWS


def _full(shape):
    return pl.BlockSpec(shape, lambda i: (0,) * len(shape))


def _rows(width):
    return pl.BlockSpec((ROWS, width), lambda i: (i, 0))


def _tc_qkv0(x3, emb, wcat):
    return pl.pallas_call(
        _qkv0_body,
        grid=(GRID,),
        in_specs=[
            pl.BlockSpec((1, 1, ROWS), lambda i: (i, 0, 0)),
            _full((128, D)),
            _full((D, 3 * D)),
        ],
        out_specs=[_rows(D), _rows(D), _rows(D), _rows(D)],
        out_shape=[jax.ShapeDtypeStruct((N, D), jnp.float32)] +
        [jax.ShapeDtypeStruct((N, D), jnp.bfloat16)] * 3,
    )(x3, emb, wcat)


def _tc_qkv1(h, wcat):
    return pl.pallas_call(
        _qkv1_body,
        grid=(GRID,),
        in_specs=[_rows(D), _full((D, 3 * D))],
        out_specs=[_rows(D), _rows(D), _rows(D)],
        out_shape=[jax.ShapeDtypeStruct((N, D), jnp.bfloat16)] * 3,
    )(h, wcat)


def _pack16(a):
    return lax.bitcast_convert_type(
        a.reshape(N, a.shape[1] // 2, 2), jnp.int32)


def _tc_post(h, num, den, wo, g1, b1, g2, b2, w1, bf1, w2, bf2):
    return pl.pallas_call(
        _post_body,
        grid=(GRID,),
        in_specs=[
            _rows(D), _rows(D), _rows(16), _full((D, D)),
            _full((1, D)), _full((1, D)), _full((1, D)), _full((1, D)),
            _full((D, FF)), _full((1, FF)), _full((FF, D)), _full((1, D)),
        ],
        out_specs=_rows(D),
        out_shape=jax.ShapeDtypeStruct((N, D), jnp.float32),
    )(h, num, den, wo, g1.reshape(1, D), b1.reshape(1, D),
      g2.reshape(1, D), b2.reshape(1, D), w1, bf1.reshape(1, FF), w2,
      bf2.reshape(1, D))


def _tc_combine(parts):
    return pl.pallas_call(
        _combine_body,
        out_shape=jax.ShapeDtypeStruct((B, D), jnp.float32),
    )(parts)


def _layer(h, src, dst_parts, wqk, wv, wo, g1, b1, g2, b2, w1, bf1, w2, bf2,
           x3=None, emb=None):
    dloc, gsrc = dst_parts
    wcat = jnp.concatenate([wqk, wv], axis=1)
    if x3 is not None:
        h, q, k, v = _tc_qkv0(x3, emb, wcat)
    else:
        q, k, v = _tc_qkv1(h, wcat)
    num, den = _edge_attention(_pack16(q), _pack16(k), _pack16(v),
                               dloc, gsrc)
    return _tc_post(h, num[:N], den[:N], wo, g1, b1, g2, b2, w1, bf1, w2,
                    bf2)


def kernel(x, complete_edge_index, ptr, batch, emb,
           W_qk_0, W_v_0, W_o_0, ln1_g_0, ln1_b_0, ln2_g_0, ln2_b_0,
           W1_0, b1_0, W2_0, b2_0,
           W_qk_1, W_v_1, W_o_1, ln1_g_1, ln1_b_1, ln2_g_1, ln2_b_1,
           W1_1, b1_1, W2_1, b2_1):
    src = complete_edge_index[0].astype(jnp.int32)
    dst = complete_edge_index[1].astype(jnp.int32)
    parts = _edge_partition(dst, src)
    x3 = x.astype(jnp.int32).reshape(GRID, 1, ROWS)
    h = _layer(None, src, parts, W_qk_0, W_v_0, W_o_0, ln1_g_0, ln1_b_0,
               ln2_g_0, ln2_b_0, W1_0, b1_0, W2_0, b2_0, x3=x3, emb=emb)
    h = _layer(h, src, parts, W_qk_1, W_v_1, W_o_1, ln1_g_1, ln1_b_1,
               ln2_g_1, ln2_b_1, W1_1, b1_1, W2_1, b2_1)
    partials = _batch_max(h, batch.astype(jnp.int32))
    return _tc_combine(partials)
